# baseline (speedup 1.0000x reference)
import jax
import jax.numpy as jnp
from jax import lax
from jax.experimental import pallas as pl
from jax.experimental.pallas import tpu as pltpu
from jax.experimental.pallas import tpu_sc as plsc

_NC = 2
_NS = 16
_NW = _NC * _NS

_XN = 10000
_XR = 312
_XTAIL = _XN - _NW * _XR
_ER = 160000 // _NW


def _sc_copy(x_hbm, e_hbm, x_out, e_out):
    wid = lax.axis_index("s") * _NC + lax.axis_index("c")
    xo = wid * _XR
    pltpu.sync_copy(x_hbm.at[pl.ds(xo, _XR)], x_out.at[pl.ds(xo, _XR)])
    eo = wid * _ER
    pltpu.sync_copy(e_hbm.at[pl.ds(eo, _ER)], e_out.at[pl.ds(eo, _ER)])

    @pl.when(wid == 0)
    def _():
        pltpu.sync_copy(x_hbm.at[pl.ds(_NW * _XR, _XTAIL)],
                        x_out.at[pl.ds(_NW * _XR, _XTAIL)])


def kernel(x, x_lstm, encoded_z_gnss, edge_index, edge_attr):
    copy = pl.kernel(
        _sc_copy,
        out_type=(
            jax.ShapeDtypeStruct(x.shape, x.dtype),
            jax.ShapeDtypeStruct(edge_attr.shape, edge_attr.dtype),
        ),
        mesh=plsc.VectorSubcoreMesh(
            core_axis_name="c", subcore_axis_name="s",
            num_cores=_NC, num_subcores=_NS,
        ),
    )
    return copy(x, edge_attr)


# speedup vs baseline: 19.5285x; 19.5285x over previous
import jax
import jax.numpy as jnp
from jax.experimental import pallas as pl
from jax.experimental.pallas import tpu as pltpu


def _copy_body(in_ref, out_ref):
    out_ref[...] = in_ref[...]


def _copy(arr, rows_per_block):
    n, d = arr.shape
    grid = n // rows_per_block
    return pl.pallas_call(
        _copy_body,
        grid=(grid,),
        out_shape=jax.ShapeDtypeStruct(arr.shape, arr.dtype),
        in_specs=[pl.BlockSpec((rows_per_block, d), lambda i: (i, 0))],
        out_specs=pl.BlockSpec((rows_per_block, d), lambda i: (i, 0)),
        compiler_params=pltpu.CompilerParams(
            dimension_semantics=("arbitrary",),
        ),
    )(arr)


def kernel(x, x_lstm, encoded_z_gnss, edge_index, edge_attr):
    x_out = _copy(x, 1000)
    e_out = _copy(edge_attr, 16000)
    return (x_out, e_out)
